# baseline (speedup 1.0000x reference)
import functools

import jax
import jax.numpy as jnp
import numpy as np
from jax import lax
from jax.experimental import pallas as pl
from jax.experimental.pallas import tpu as pltpu
from jax.experimental.pallas import tpu_sc as plsc

NUM_NODE = 100000
DIM = 64
B = 256
L = 50
LP = 56
N_SAMPLE = 12
ALPHA = 0.2
BETA = 0.005
NEG = -9e15

NC, NS = 2, 16
NW = NC * NS
BL = B * L
NBRP = B * N_SAMPLE * LP
SPAD = 16

_PB = np.array([56, 119, 206, 213, 133, 249, 174, 246, 111, 105, 96, 166, 61, 195, 127, 117, 188, 103, 122, 220, 201, 128, 73, 3, 97, 38, 224, 45, 107, 196, 210, 93, 162, 82, 157, 86, 8, 114, 155, 245, 74, 15, 9, 88, 42, 78, 52, 16, 125, 20, 247, 187, 163, 83, 255, 228, 35, 253, 191, 207, 101, 39, 106, 156, 118, 152, 129, 237, 190, 136, 49, 139, 50, 89, 151, 184, 172, 150, 153, 175, 113, 44, 51, 55, 192, 169, 244, 251, 205, 209, 130, 84, 126, 193, 29, 137, 202, 138, 146, 200, 0, 22, 177, 11, 17, 243, 18, 148, 4, 67, 69, 21, 66, 215, 72, 23, 77, 81, 32, 40, 108, 197, 199, 218, 91, 12, 140, 189, 231, 110, 24, 31, 154, 159, 43, 241, 2, 48, 248, 124, 145, 233, 214, 225, 170, 171, 131, 27, 198, 158, 238, 79, 142, 242, 164, 30, 34, 234, 58, 115, 41, 112, 161, 235, 204, 98, 36, 167, 144, 135, 227, 53, 63, 208, 10, 68, 132, 120, 252, 160, 165, 47, 71, 223, 104, 181, 141, 5, 229, 13, 87, 194, 250, 28, 121, 254, 100, 217, 239, 57, 180, 222, 70, 211, 109, 212, 90, 143, 6, 60, 37, 33, 183, 182, 123, 232, 14, 176, 226, 95, 134, 230, 186, 46, 85, 19, 179, 173, 147, 94, 76, 65, 216, 80, 185, 240, 99, 178, 236, 25, 219, 102, 116, 7, 54, 75, 149, 59, 203, 221, 1, 62, 64, 26, 92, 168], dtype=np.int32)
_PL = np.array([1, 47, 23, 19, 33, 2, 42, 16, 40, 39, 3, 8, 41, 48, 7, 31, 18, 32, 49, 22, 27, 15, 29, 44, 5, 17, 24, 6, 12, 10, 37, 35, 38, 4, 14, 0, 13, 46, 25, 11, 20, 30, 36, 28, 26, 34, 43, 9, 45, 21], dtype=np.int32)
_PB_MAT = np.zeros((B, B), np.float32)
_PB_MAT[np.arange(B), _PB] = 1.0


def _leaky(x, slope):
    return jnp.maximum(x, slope * x)


def _sc_gather_emb(embedding, idx_in, idx_item, idx_tgt):
    n_per_w = BL // NW
    ch = 80
    n_ch = n_per_w // ch
    t_per_w = B // NW
    mesh = plsc.VectorSubcoreMesh(core_axis_name="c", subcore_axis_name="s")

    @functools.partial(
        pl.kernel,
        mesh=mesh,
        compiler_params=pltpu.CompilerParams(use_tc_tiling_on_sc=False),
        out_type=(
            jax.ShapeDtypeStruct((BL, DIM), jnp.float32),
            jax.ShapeDtypeStruct((BL, DIM), jnp.float32),
            jax.ShapeDtypeStruct((B, DIM), jnp.float32),
        ),
        scratch_types=[
            pltpu.VMEM((n_per_w,), jnp.int32),
            pltpu.VMEM((n_per_w,), jnp.int32),
            pltpu.VMEM((ch, DIM), jnp.float32),
            pltpu.VMEM((ch, DIM), jnp.float32),
            pltpu.VMEM((t_per_w,), jnp.int32),
            pltpu.VMEM((t_per_w, DIM), jnp.float32),
            pltpu.SemaphoreType.DMA,
            pltpu.SemaphoreType.DMA,
        ],
    )
    def k(emb_hbm, iin_hbm, iit_hbm, itg_hbm,
          h_hbm, item_hbm, t1_hbm,
          iin_v, iit_v, rows_a, rows_b, tidx_v, trows_v, sem_a, sem_b):
        wid = lax.axis_index("s") * NC + lax.axis_index("c")
        base0 = wid * n_per_w
        pltpu.sync_copy(iin_hbm.at[pl.ds(base0, n_per_w)], iin_v)
        pltpu.sync_copy(iit_hbm.at[pl.ds(base0, n_per_w)], iit_v)

        @pl.loop(0, n_ch)
        def _(c):
            sl = pl.ds(base0 + c * ch, ch)
            islc = pl.ds(c * ch, ch)
            pltpu.async_copy(emb_hbm.at[iin_v.at[islc]], rows_a, sem_a).wait()
            pltpu.async_copy(emb_hbm.at[iit_v.at[islc]], rows_b, sem_b).wait()
            pltpu.sync_copy(rows_a, h_hbm.at[sl])
            pltpu.sync_copy(rows_b, item_hbm.at[sl])

        tsl = pl.ds(wid * t_per_w, t_per_w)
        pltpu.sync_copy(itg_hbm.at[tsl], tidx_v)
        pltpu.async_copy(emb_hbm.at[tidx_v], trows_v, sem_a).wait()
        pltpu.sync_copy(trows_v, t1_hbm.at[tsl])

    return k(embedding, idx_in, idx_item, idx_tgt)


def _sc_gather_tables(catp, idx_in):
    n_per_w = BL // NW
    ch = 80
    n_ch = n_per_w // ch
    mesh = plsc.VectorSubcoreMesh(core_axis_name="c", subcore_axis_name="s")

    @functools.partial(
        pl.kernel,
        mesh=mesh,
        compiler_params=pltpu.CompilerParams(use_tc_tiling_on_sc=False),
        out_type=jax.ShapeDtypeStruct((BL, 2 * SPAD), jnp.int32),
        scratch_types=[
            pltpu.VMEM((n_per_w,), jnp.int32),
            pltpu.VMEM((ch, 2 * SPAD), jnp.int32),
            pltpu.SemaphoreType.DMA,
        ],
    )
    def k(cat_hbm, iin_hbm, out_hbm, iin_v, rows_v, sem):
        wid = lax.axis_index("s") * NC + lax.axis_index("c")
        base0 = wid * n_per_w
        pltpu.sync_copy(iin_hbm.at[pl.ds(base0, n_per_w)], iin_v)

        @pl.loop(0, n_ch)
        def _(c):
            sl = pl.ds(base0 + c * ch, ch)
            pltpu.async_copy(cat_hbm.at[iin_v.at[pl.ds(c * ch, ch)]],
                             rows_v, sem).wait()
            pltpu.sync_copy(rows_v, out_hbm.at[sl])

    return k(catp, idx_in)


def _sc_gather_level2(embedding, nbr_idx, nrows):
    n_per_w = nrows // NW
    ch = 112
    n_ch = n_per_w // ch
    mesh = plsc.VectorSubcoreMesh(core_axis_name="c", subcore_axis_name="s")

    @functools.partial(
        pl.kernel,
        mesh=mesh,
        compiler_params=pltpu.CompilerParams(use_tc_tiling_on_sc=False),
        out_type=jax.ShapeDtypeStruct((nrows, DIM), jnp.float32),
        scratch_types=[
            pltpu.VMEM((ch,), jnp.int32),
            pltpu.VMEM((ch, DIM), jnp.float32),
            pltpu.SemaphoreType.DMA,
        ],
    )
    def k(emb_hbm, idx_hbm, out_hbm, idx_v, rows_v, sem):
        wid = lax.axis_index("s") * NC + lax.axis_index("c")
        base0 = wid * n_per_w

        @pl.loop(0, n_ch)
        def _(c):
            sl = pl.ds(base0 + c * ch, ch)
            pltpu.sync_copy(idx_hbm.at[sl], idx_v)
            pltpu.async_copy(emb_hbm.at[idx_v], rows_v, sem).wait()
            pltpu.sync_copy(rows_v, out_hbm.at[sl])

    return k(embedding, nbr_idx)


_BB = 16


def _dense_body(h_ref, adj_ref, mask_ref, t1_ref, wn_ref, nbr_ref, item_ref,
                acat_ref, taw_ref, tav_ref, gw1_ref, gw2_ref, gw3_ref,
                out_ref, hl_ref, hg_ref):
    f32 = jnp.float32
    mask3 = mask_ref[...]
    h_blk = h_ref[...]
    te = jnp.dot(t1_ref[...], taw_ref[DIM:2 * DIM, :],
                 preferred_element_type=f32)
    sess = (jnp.sum(item_ref[...] * mask3, axis=1)
            / jnp.sum(mask3, axis=1))

    es = []
    for b in range(_BB):
        hb = h_blk[b]
        hs4 = jnp.concatenate(
            [hb * acat_ref[k4:k4 + 1, :] for k4 in range(4)], axis=0)
        eb = lax.dot_general(hs4, hb, (((1,), (1,)), ((), ())),
                             preferred_element_type=f32)
        es.append(eb.reshape(1, 4, LP, LP))
    e_all = jnp.concatenate(es, axis=0)
    adjb = adj_ref[...]
    att = jnp.full((_BB, LP, LP), NEG, f32)
    att = jnp.where(adjb == 1, _leaky(e_all[:, 0], ALPHA), att)
    att = jnp.where(adjb == 2, _leaky(e_all[:, 1], ALPHA), att)
    att = jnp.where(adjb == 3, _leaky(e_all[:, 2], ALPHA), att)
    att = jnp.where(adjb == 4, _leaky(e_all[:, 3], ALPHA), att)
    att = jax.nn.softmax(att, axis=-1)
    hls = []
    for b in range(_BB):
        hlb = jnp.dot(att[b], h_blk[b], preferred_element_type=f32)
        hls.append(hlb.reshape(1, LP, DIM))
    hl_all = jnp.concatenate(hls, axis=0)

    hf = h_blk.reshape(_BB * LP, DIM)
    e = _leaky(jnp.dot(hf, taw_ref[0:DIM, :], preferred_element_type=f32)
               + jnp.broadcast_to(te[:, None, :], (_BB, LP, DIM))
               .reshape(_BB * LP, DIM), ALPHA)
    score = jnp.sum(e * tav_ref[...], axis=-1, keepdims=True)
    score = score.reshape(_BB, LP, 1)
    score = jnp.where(mask3 > 0, score, NEG)
    alpha = jax.nn.softmax(score, axis=1)
    ht_all = alpha * h_blk

    nbrf = nbr_ref[...]
    nbr4 = nbrf.reshape(_BB, N_SAMPLE, LP, DIM)
    sessx = jnp.broadcast_to(sess[:, None, None, :],
                             (_BB, N_SAMPLE, LP, DIM))
    ex = sessx.reshape(_BB * N_SAMPLE * LP, DIM) * nbrf
    p = jnp.dot(ex, gw1_ref[0:DIM, :], preferred_element_type=f32)
    p = p.reshape(_BB, N_SAMPLE, LP, DIM) + wn_ref[...] * gw1_ref[DIM:DIM + 1, :]
    p = _leaky(p, 0.2)
    sc2 = jnp.sum(p * gw2_ref[...], axis=-1, keepdims=True)
    m2 = jnp.max(sc2, axis=1, keepdims=True)
    ex2 = jnp.exp(sc2 - m2)
    att2 = ex2 / jnp.sum(ex2, axis=1, keepdims=True)
    neigh = jnp.sum(att2 * nbr4, axis=1)
    hg_all = jnp.maximum(
        jnp.dot(hf, gw3_ref[0:DIM, :], preferred_element_type=f32)
        + jnp.dot(neigh.reshape(_BB * LP, DIM), gw3_ref[DIM:2 * DIM, :],
                  preferred_element_type=f32), 0.0).reshape(_BB, LP, DIM)

    out_ref[:, 0:L, :] = ht_all[:, 0:L, :]
    out_ref[:, L:2 * L, :] = hl_all[:, 0:L, :] + hg_all[:, 0:L, :]
    hl_ref[...] = hl_all[:, 0:L, :]
    hg_ref[...] = hg_all[:, 0:L, :]


def _loss_body(hla_ref, hlb_ref, hga_ref, hgb_ref, pb_ref, out_ref):
    f32 = jnp.float32
    hl3 = jnp.concatenate([hla_ref[...], hlb_ref[...]], axis=0)
    hg3 = jnp.concatenate([hga_ref[...], hgb_ref[...]], axis=0)
    pb = pb_ref[...]
    pos = jnp.sum(hl3 * hg3, axis=1)
    neg = jnp.zeros((B, DIM), f32)
    for l in range(L):
        cl = jnp.dot(pb, hl3[:, int(_PL[l]), :], preferred_element_type=f32)
        neg = neg + hg3[:, l, :] * cl
    spos = jax.nn.sigmoid(pos)
    sneg = jax.nn.sigmoid(neg)
    total = jnp.sum(-jnp.log(1e-8 + spos) - jnp.log(1e-8 + (1.0 - sneg)))
    out_ref[...] = total[None, None]


def kernel(inputs, adj, mask_item, item, targets, adj_all, num_w, embedding,
           la_a0, la_a1, la_a2, la_a3, ta_w, ta_v, ga_w1, ga_w2, ga_w3):
    idx_in = inputs.reshape(-1).astype(jnp.int32)
    idx_item = item.reshape(-1).astype(jnp.int32)
    idx_tgt = targets.astype(jnp.int32)

    catp = jnp.concatenate(
        [adj_all.astype(jnp.int32),
         jax.lax.bitcast_convert_type(num_w, jnp.int32),
         jnp.zeros((NUM_NODE, 2 * SPAD - 2 * N_SAMPLE), jnp.int32)], axis=1)
    h_flat, item_flat, t1 = _sc_gather_emb(embedding, idx_in, idx_item, idx_tgt)
    samp_wn = _sc_gather_tables(catp, idx_in)
    samp = samp_wn[:, :N_SAMPLE]
    wn = jax.lax.bitcast_convert_type(
        samp_wn[:, N_SAMPLE:2 * N_SAMPLE], jnp.float32)

    nbr_idx = samp.reshape(B, L, N_SAMPLE).transpose(0, 2, 1)
    nbr_idx = jnp.concatenate([nbr_idx, nbr_idx[..., :LP - L]], axis=2)
    nbr_idx = nbr_idx.reshape(-1)
    nbr_a = _sc_gather_level2(embedding, nbr_idx[:NBRP // 2], NBRP // 2)
    nbr_b = _sc_gather_level2(embedding, nbr_idx[NBRP // 2:], NBRP // 2)

    pad_l = ((0, 0), (0, LP - L), (0, 0))
    h = jnp.pad(h_flat.reshape(B, L, DIM), pad_l)
    item_emb = jnp.pad(item_flat.reshape(B, L, DIM), pad_l)
    adj_p = jnp.pad(adj, ((0, 0), (0, LP - L), (0, LP - L)))
    mask3 = jnp.pad(mask_item.astype(jnp.float32).reshape(B, L, 1), pad_l)
    wn4 = jnp.pad(wn.reshape(B, L, N_SAMPLE).transpose(0, 2, 1),
                  ((0, 0), (0, 0), (0, LP - L)))[..., None]
    acat = jnp.concatenate([la_a0, la_a1, la_a2, la_a3], axis=1).T
    tav = ta_v.T
    gw2 = ga_w2.T

    grid = B // (2 * _BB)
    hb2 = B // 2

    def dense_half(nbr_half, off):
        return pl.pallas_call(
            _dense_body,
            grid=(grid,),
            in_specs=[
                pl.BlockSpec((_BB, LP, DIM), lambda i: (i + off, 0, 0)),
                pl.BlockSpec((_BB, LP, LP), lambda i: (i + off, 0, 0)),
                pl.BlockSpec((_BB, LP, 1), lambda i: (i + off, 0, 0)),
                pl.BlockSpec((_BB, DIM), lambda i: (i + off, 0)),
                pl.BlockSpec((_BB, N_SAMPLE, LP, 1), lambda i: (i + off, 0, 0, 0)),
                pl.BlockSpec((_BB * N_SAMPLE * LP, DIM), lambda i: (i, 0)),
                pl.BlockSpec((_BB, LP, DIM), lambda i: (i + off, 0, 0)),
                pl.BlockSpec((4, DIM), lambda i: (0, 0)),
                pl.BlockSpec((2 * DIM, DIM), lambda i: (0, 0)),
                pl.BlockSpec((1, DIM), lambda i: (0, 0)),
                pl.BlockSpec((DIM + 1, DIM), lambda i: (0, 0)),
                pl.BlockSpec((1, DIM), lambda i: (0, 0)),
                pl.BlockSpec((2 * DIM, DIM), lambda i: (0, 0)),
            ],
            out_specs=[
                pl.BlockSpec((_BB, 2 * L, DIM), lambda i: (i, 0, 0)),
                pl.BlockSpec((_BB, L, DIM), lambda i: (i, 0, 0)),
                pl.BlockSpec((_BB, L, DIM), lambda i: (i, 0, 0)),
            ],
            out_shape=[
                jax.ShapeDtypeStruct((hb2, 2 * L, DIM), jnp.float32),
                jax.ShapeDtypeStruct((hb2, L, DIM), jnp.float32),
                jax.ShapeDtypeStruct((hb2, L, DIM), jnp.float32),
            ],
        )(h, adj_p, mask3, t1, wn4, nbr_half, item_emb, acat, ta_w, tav,
          ga_w1, gw2, ga_w3)

    out_a, hl_a, hg_a = dense_half(nbr_a, 0)
    out_b, hl_b, hg_b = dense_half(nbr_b, grid)

    loss = pl.pallas_call(
        _loss_body,
        out_shape=jax.ShapeDtypeStruct((1, 1), jnp.float32),
    )(hl_a, hl_b, hg_a, hg_b, jnp.asarray(_PB_MAT))

    return jnp.concatenate([out_a, out_b], axis=0), BETA * loss[0, 0]


# speedup vs baseline: 1.0395x; 1.0395x over previous
import functools

import jax
import jax.numpy as jnp
import numpy as np
from jax import lax
from jax.experimental import pallas as pl
from jax.experimental.pallas import tpu as pltpu
from jax.experimental.pallas import tpu_sc as plsc

NUM_NODE = 100000
DIM = 64
B = 256
L = 50
LP = 56
N_SAMPLE = 12
ALPHA = 0.2
BETA = 0.005
NEG = -9e15

NC, NS = 2, 16
NW = NC * NS
BL = B * L
NBRP = B * N_SAMPLE * LP
SPAD = 16

_PB = np.array([56, 119, 206, 213, 133, 249, 174, 246, 111, 105, 96, 166, 61, 195, 127, 117, 188, 103, 122, 220, 201, 128, 73, 3, 97, 38, 224, 45, 107, 196, 210, 93, 162, 82, 157, 86, 8, 114, 155, 245, 74, 15, 9, 88, 42, 78, 52, 16, 125, 20, 247, 187, 163, 83, 255, 228, 35, 253, 191, 207, 101, 39, 106, 156, 118, 152, 129, 237, 190, 136, 49, 139, 50, 89, 151, 184, 172, 150, 153, 175, 113, 44, 51, 55, 192, 169, 244, 251, 205, 209, 130, 84, 126, 193, 29, 137, 202, 138, 146, 200, 0, 22, 177, 11, 17, 243, 18, 148, 4, 67, 69, 21, 66, 215, 72, 23, 77, 81, 32, 40, 108, 197, 199, 218, 91, 12, 140, 189, 231, 110, 24, 31, 154, 159, 43, 241, 2, 48, 248, 124, 145, 233, 214, 225, 170, 171, 131, 27, 198, 158, 238, 79, 142, 242, 164, 30, 34, 234, 58, 115, 41, 112, 161, 235, 204, 98, 36, 167, 144, 135, 227, 53, 63, 208, 10, 68, 132, 120, 252, 160, 165, 47, 71, 223, 104, 181, 141, 5, 229, 13, 87, 194, 250, 28, 121, 254, 100, 217, 239, 57, 180, 222, 70, 211, 109, 212, 90, 143, 6, 60, 37, 33, 183, 182, 123, 232, 14, 176, 226, 95, 134, 230, 186, 46, 85, 19, 179, 173, 147, 94, 76, 65, 216, 80, 185, 240, 99, 178, 236, 25, 219, 102, 116, 7, 54, 75, 149, 59, 203, 221, 1, 62, 64, 26, 92, 168], dtype=np.int32)
_PL = np.array([1, 47, 23, 19, 33, 2, 42, 16, 40, 39, 3, 8, 41, 48, 7, 31, 18, 32, 49, 22, 27, 15, 29, 44, 5, 17, 24, 6, 12, 10, 37, 35, 38, 4, 14, 0, 13, 46, 25, 11, 20, 30, 36, 28, 26, 34, 43, 9, 45, 21], dtype=np.int32)
_PB_MAT = np.zeros((B, B), np.float32)
_PB_MAT[np.arange(B), _PB] = 1.0


def _leaky(x, slope):
    return jnp.maximum(x, slope * x)


def _sc_gather_emb(embedding, idx_in, idx_item, idx_tgt):
    n_per_w = BL // NW
    ch = 80
    n_ch = n_per_w // ch
    t_per_w = B // NW
    mesh = plsc.VectorSubcoreMesh(core_axis_name="c", subcore_axis_name="s")

    @functools.partial(
        pl.kernel,
        mesh=mesh,
        compiler_params=pltpu.CompilerParams(use_tc_tiling_on_sc=False),
        out_type=(
            jax.ShapeDtypeStruct((BL, DIM), jnp.float32),
            jax.ShapeDtypeStruct((BL, DIM), jnp.float32),
            jax.ShapeDtypeStruct((B, DIM), jnp.float32),
        ),
        scratch_types=[
            pltpu.VMEM((n_per_w,), jnp.int32),
            pltpu.VMEM((n_per_w,), jnp.int32),
            pltpu.VMEM((ch, DIM), jnp.float32),
            pltpu.VMEM((ch, DIM), jnp.float32),
            pltpu.VMEM((t_per_w,), jnp.int32),
            pltpu.VMEM((t_per_w, DIM), jnp.float32),
            pltpu.SemaphoreType.DMA,
            pltpu.SemaphoreType.DMA,
        ],
    )
    def k(emb_hbm, iin_hbm, iit_hbm, itg_hbm,
          h_hbm, item_hbm, t1_hbm,
          iin_v, iit_v, rows_a, rows_b, tidx_v, trows_v, sem_a, sem_b):
        wid = lax.axis_index("s") * NC + lax.axis_index("c")
        base0 = wid * n_per_w
        pltpu.sync_copy(iin_hbm.at[pl.ds(base0, n_per_w)], iin_v)
        pltpu.sync_copy(iit_hbm.at[pl.ds(base0, n_per_w)], iit_v)

        @pl.loop(0, n_ch)
        def _(c):
            sl = pl.ds(base0 + c * ch, ch)
            islc = pl.ds(c * ch, ch)
            pltpu.async_copy(emb_hbm.at[iin_v.at[islc]], rows_a, sem_a).wait()
            pltpu.async_copy(emb_hbm.at[iit_v.at[islc]], rows_b, sem_b).wait()
            pltpu.sync_copy(rows_a, h_hbm.at[sl])
            pltpu.sync_copy(rows_b, item_hbm.at[sl])

        tsl = pl.ds(wid * t_per_w, t_per_w)
        pltpu.sync_copy(itg_hbm.at[tsl], tidx_v)
        pltpu.async_copy(emb_hbm.at[tidx_v], trows_v, sem_a).wait()
        pltpu.sync_copy(trows_v, t1_hbm.at[tsl])

    return k(embedding, idx_in, idx_item, idx_tgt)


def _sc_gather_tables(catp, idx_in):
    n_per_w = BL // NW
    ch = 80
    n_ch = n_per_w // ch
    mesh = plsc.VectorSubcoreMesh(core_axis_name="c", subcore_axis_name="s")

    @functools.partial(
        pl.kernel,
        mesh=mesh,
        compiler_params=pltpu.CompilerParams(use_tc_tiling_on_sc=False),
        out_type=jax.ShapeDtypeStruct((BL, 2 * SPAD), jnp.int32),
        scratch_types=[
            pltpu.VMEM((n_per_w,), jnp.int32),
            pltpu.VMEM((ch, 2 * SPAD), jnp.int32),
            pltpu.SemaphoreType.DMA,
        ],
    )
    def k(cat_hbm, iin_hbm, out_hbm, iin_v, rows_v, sem):
        wid = lax.axis_index("s") * NC + lax.axis_index("c")
        base0 = wid * n_per_w
        pltpu.sync_copy(iin_hbm.at[pl.ds(base0, n_per_w)], iin_v)

        @pl.loop(0, n_ch)
        def _(c):
            sl = pl.ds(base0 + c * ch, ch)
            pltpu.async_copy(cat_hbm.at[iin_v.at[pl.ds(c * ch, ch)]],
                             rows_v, sem).wait()
            pltpu.sync_copy(rows_v, out_hbm.at[sl])

    return k(catp, idx_in)


def _sc_gather_level2(embedding, nbr_idx, nrows):
    n_per_w = nrows // NW
    ch = 112
    n_ch = n_per_w // ch
    mesh = plsc.VectorSubcoreMesh(core_axis_name="c", subcore_axis_name="s")

    @functools.partial(
        pl.kernel,
        mesh=mesh,
        compiler_params=pltpu.CompilerParams(use_tc_tiling_on_sc=False),
        out_type=jax.ShapeDtypeStruct((nrows, DIM), jnp.float32),
        scratch_types=[
            pltpu.VMEM((ch,), jnp.int32),
            pltpu.VMEM((ch, DIM), jnp.float32),
            pltpu.SemaphoreType.DMA,
        ],
    )
    def k(emb_hbm, idx_hbm, out_hbm, idx_v, rows_v, sem):
        wid = lax.axis_index("s") * NC + lax.axis_index("c")
        base0 = wid * n_per_w

        @pl.loop(0, n_ch)
        def _(c):
            sl = pl.ds(base0 + c * ch, ch)
            pltpu.sync_copy(idx_hbm.at[sl], idx_v)
            pltpu.async_copy(emb_hbm.at[idx_v], rows_v, sem).wait()
            pltpu.sync_copy(rows_v, out_hbm.at[sl])

    return k(embedding, nbr_idx)


_BB = 16


def _dense_body(h_ref, adj_ref, mask_ref, t1_ref, wn_ref, nbr_ref, item_ref,
                acat_ref, taw_ref, tav_ref, gw1_ref, gw2_ref, gw3_ref,
                out_ref, hl_ref, hg_ref):
    f32 = jnp.float32
    mask3 = mask_ref[...]
    h_blk = h_ref[...]
    te = jnp.dot(t1_ref[...], taw_ref[DIM:2 * DIM, :],
                 preferred_element_type=f32)
    sess = (jnp.sum(item_ref[...] * mask3, axis=1)
            / jnp.sum(mask3, axis=1))

    es = []
    for b in range(_BB):
        hb = h_blk[b]
        hs4 = jnp.concatenate(
            [hb * acat_ref[k4:k4 + 1, :] for k4 in range(4)], axis=0)
        eb = lax.dot_general(hs4, hb, (((1,), (1,)), ((), ())),
                             preferred_element_type=f32)
        es.append(eb.reshape(1, 4, LP, LP))
    e_all = jnp.concatenate(es, axis=0)
    adjb = adj_ref[...]
    att = jnp.full((_BB, LP, LP), NEG, f32)
    att = jnp.where(adjb == 1, _leaky(e_all[:, 0], ALPHA), att)
    att = jnp.where(adjb == 2, _leaky(e_all[:, 1], ALPHA), att)
    att = jnp.where(adjb == 3, _leaky(e_all[:, 2], ALPHA), att)
    att = jnp.where(adjb == 4, _leaky(e_all[:, 3], ALPHA), att)
    att = jax.nn.softmax(att, axis=-1)
    hls = []
    for b in range(_BB):
        hlb = jnp.dot(att[b], h_blk[b], preferred_element_type=f32)
        hls.append(hlb.reshape(1, LP, DIM))
    hl_all = jnp.concatenate(hls, axis=0)

    hf = h_blk.reshape(_BB * LP, DIM)
    e = _leaky(jnp.dot(hf, taw_ref[0:DIM, :], preferred_element_type=f32)
               + jnp.broadcast_to(te[:, None, :], (_BB, LP, DIM))
               .reshape(_BB * LP, DIM), ALPHA)
    score = jnp.sum(e * tav_ref[...], axis=-1, keepdims=True)
    score = score.reshape(_BB, LP, 1)
    score = jnp.where(mask3 > 0, score, NEG)
    alpha = jax.nn.softmax(score, axis=1)
    ht_all = alpha * h_blk

    nbrf = nbr_ref[...]
    nbr4 = nbrf.reshape(_BB, N_SAMPLE, LP, DIM)
    sessx = jnp.broadcast_to(sess[:, None, None, :],
                             (_BB, N_SAMPLE, LP, DIM))
    ex = sessx.reshape(_BB * N_SAMPLE * LP, DIM) * nbrf
    p = jnp.dot(ex, gw1_ref[0:DIM, :], preferred_element_type=f32)
    p = p.reshape(_BB, N_SAMPLE, LP, DIM) + wn_ref[...] * gw1_ref[DIM:DIM + 1, :]
    p = _leaky(p, 0.2)
    sc2 = jnp.sum(p * gw2_ref[...], axis=-1, keepdims=True)
    m2 = jnp.max(sc2, axis=1, keepdims=True)
    ex2 = jnp.exp(sc2 - m2)
    att2 = ex2 / jnp.sum(ex2, axis=1, keepdims=True)
    neigh = jnp.sum(att2 * nbr4, axis=1)
    hg_all = jnp.maximum(
        jnp.dot(hf, gw3_ref[0:DIM, :], preferred_element_type=f32)
        + jnp.dot(neigh.reshape(_BB * LP, DIM), gw3_ref[DIM:2 * DIM, :],
                  preferred_element_type=f32), 0.0).reshape(_BB, LP, DIM)

    out_ref[:, 0:L, :] = ht_all[:, 0:L, :]
    out_ref[:, L:2 * L, :] = hl_all[:, 0:L, :] + hg_all[:, 0:L, :]
    hl_ref[...] = hl_all[:, 0:L, :]
    hg_ref[...] = hg_all[:, 0:L, :]


def _loss_body(hl_ref, hg_ref, pb_ref, out_ref):
    f32 = jnp.float32
    hl3 = hl_ref[...]
    hg3 = hg_ref[...]
    pb = pb_ref[...]
    pos = jnp.sum(hl3 * hg3, axis=1)
    neg = jnp.zeros((B, DIM), f32)
    for l in range(L):
        cl = jnp.dot(pb, hl3[:, int(_PL[l]), :], preferred_element_type=f32)
        neg = neg + hg3[:, l, :] * cl
    spos = jax.nn.sigmoid(pos)
    sneg = jax.nn.sigmoid(neg)
    total = jnp.sum(-jnp.log(1e-8 + spos) - jnp.log(1e-8 + (1.0 - sneg)))
    out_ref[...] = total[None, None]


def kernel(inputs, adj, mask_item, item, targets, adj_all, num_w, embedding,
           la_a0, la_a1, la_a2, la_a3, ta_w, ta_v, ga_w1, ga_w2, ga_w3):
    idx_in = inputs.reshape(-1).astype(jnp.int32)
    idx_item = item.reshape(-1).astype(jnp.int32)
    idx_tgt = targets.astype(jnp.int32)

    catp = jnp.concatenate(
        [adj_all.astype(jnp.int32),
         jax.lax.bitcast_convert_type(num_w, jnp.int32),
         jnp.zeros((NUM_NODE, 2 * SPAD - 2 * N_SAMPLE), jnp.int32)], axis=1)
    h_flat, item_flat, t1 = _sc_gather_emb(embedding, idx_in, idx_item, idx_tgt)
    samp_wn = _sc_gather_tables(catp, idx_in)
    samp = samp_wn[:, :N_SAMPLE]
    wn = jax.lax.bitcast_convert_type(
        samp_wn[:, N_SAMPLE:2 * N_SAMPLE], jnp.float32)

    nbr_idx = samp.reshape(B, L, N_SAMPLE).transpose(0, 2, 1)
    nbr_idx = jnp.concatenate([nbr_idx, nbr_idx[..., :LP - L]], axis=2)
    nbr_flat = _sc_gather_level2(embedding, nbr_idx.reshape(-1), NBRP)

    pad_l = ((0, 0), (0, LP - L), (0, 0))
    h = jnp.pad(h_flat.reshape(B, L, DIM), pad_l)
    item_emb = jnp.pad(item_flat.reshape(B, L, DIM), pad_l)
    adj_p = jnp.pad(adj, ((0, 0), (0, LP - L), (0, LP - L)))
    mask3 = jnp.pad(mask_item.astype(jnp.float32).reshape(B, L, 1), pad_l)
    wn4 = jnp.pad(wn.reshape(B, L, N_SAMPLE).transpose(0, 2, 1),
                  ((0, 0), (0, 0), (0, LP - L)))[..., None]
    acat = jnp.concatenate([la_a0, la_a1, la_a2, la_a3], axis=1).T
    tav = ta_v.T
    gw2 = ga_w2.T

    grid = B // _BB
    out, hl, hg = pl.pallas_call(
        _dense_body,
        grid=(grid,),
        in_specs=[
            pl.BlockSpec((_BB, LP, DIM), lambda i: (i, 0, 0)),
            pl.BlockSpec((_BB, LP, LP), lambda i: (i, 0, 0)),
            pl.BlockSpec((_BB, LP, 1), lambda i: (i, 0, 0)),
            pl.BlockSpec((_BB, DIM), lambda i: (i, 0)),
            pl.BlockSpec((_BB, N_SAMPLE, LP, 1), lambda i: (i, 0, 0, 0)),
            pl.BlockSpec((_BB * N_SAMPLE * LP, DIM), lambda i: (i, 0)),
            pl.BlockSpec((_BB, LP, DIM), lambda i: (i, 0, 0)),
            pl.BlockSpec((4, DIM), lambda i: (0, 0)),
            pl.BlockSpec((2 * DIM, DIM), lambda i: (0, 0)),
            pl.BlockSpec((1, DIM), lambda i: (0, 0)),
            pl.BlockSpec((DIM + 1, DIM), lambda i: (0, 0)),
            pl.BlockSpec((1, DIM), lambda i: (0, 0)),
            pl.BlockSpec((2 * DIM, DIM), lambda i: (0, 0)),
        ],
        out_specs=[
            pl.BlockSpec((_BB, 2 * L, DIM), lambda i: (i, 0, 0)),
            pl.BlockSpec((_BB, L, DIM), lambda i: (i, 0, 0)),
            pl.BlockSpec((_BB, L, DIM), lambda i: (i, 0, 0)),
        ],
        out_shape=[
            jax.ShapeDtypeStruct((B, 2 * L, DIM), jnp.float32),
            jax.ShapeDtypeStruct((B, L, DIM), jnp.float32),
            jax.ShapeDtypeStruct((B, L, DIM), jnp.float32),
        ],
    )(h, adj_p, mask3, t1, wn4, nbr_flat, item_emb, acat, ta_w, tav, ga_w1,
      gw2, ga_w3)

    loss = pl.pallas_call(
        _loss_body,
        out_shape=jax.ShapeDtypeStruct((1, 1), jnp.float32),
    )(hl, hg, jnp.asarray(_PB_MAT))

    return out, BETA * loss[0, 0]
